# baseline (speedup 1.0000x reference)
import functools

import jax
import jax.numpy as jnp
import numpy as np
from jax import lax
from jax.experimental import pallas as pl
from jax.experimental.pallas import tpu as pltpu
from jax.experimental.pallas import tpu_sc as plsc

N = 10000
E = 160000
D = 256
H = 8
R = 6
NS = 16
NC = 2

_mesh = functools.partial(
    plsc.VectorSubcoreMesh, core_axis_name="c", subcore_axis_name="s"
)


def _make_segsum(W, table_rows, splits, C, npass):
    EPT = E // NS
    Cacc = C + NS
    CAP = EPT + 368
    ZB = 32
    KEYS = splits[-1][1]
    assert Cacc % NS == 0 and EPT % 16 == 0

    @functools.partial(
        pl.kernel,
        mesh=_mesh(),
        compiler_params=pltpu.CompilerParams(use_tc_tiling_on_sc=False, needs_layout_passes=False),
        out_type=jax.ShapeDtypeStruct((KEYS, W), jnp.float32),
        scratch_types=[
            pltpu.VMEM((EPT,), jnp.int32),
            pltpu.VMEM((EPT,), jnp.int32),
            pltpu.VMEM((CAP,), jnp.int32),
            pltpu.VMEM((CAP // 32, 32), jnp.int32),
            pltpu.VMEM((64, W), jnp.float32),
            pltpu.VMEM((64, W), jnp.float32),
            pltpu.VMEM((ZB, W), jnp.float32),
            pltpu.VMEM_SHARED((Cacc, W), jnp.float32),
            pltpu.SemaphoreType.DMA,
            pltpu.SemaphoreType.DMA,
            pltpu.SemaphoreType.DMA,
            pltpu.SemaphoreType.DMA,
            pltpu.SemaphoreType.DMA,
            pltpu.SemaphoreType.DMA,
            pltpu.SemaphoreType.DMA,
        ],
    )
    def seg(table_hbm, gv_hbm, key_hbm, out_hbm,
            gvals, keys, cg, ck, rows0, rows1, zbuf, accum,
            g0a, g0b, g1a, g1b, t0, t1, zsem):
        c = lax.axis_index("c")
        s = lax.axis_index("s")
        ebase = s * EPT
        pltpu.sync_copy(gv_hbm.at[pl.ds(ebase, EPT)], gvals)
        pltpu.sync_copy(key_hbm.at[pl.ds(ebase, EPT)], keys)

        zv = jnp.zeros((16,), jnp.float32)
        wl = W // 16

        def zb_body(i, carry):
            zbuf[i // wl, pl.ds((i % wl) * 16, 16)] = zv
            return carry

        lax.fori_loop(0, ZB * wl, zb_body, 0)

        zrows = Cacc // NS
        zlo = s * zrows

        def gdesc(b, rbuf, sa, sb):
            return (
                pltpu.make_async_copy(
                    table_hbm.at[cg.at[pl.ds(b * 64, 32)]],
                    rbuf.at[pl.ds(0, 32)], sa),
                pltpu.make_async_copy(
                    table_hbm.at[cg.at[pl.ds(b * 64 + 32, 32)]],
                    rbuf.at[pl.ds(32, 32)], sb),
            )

        def gstart(b, rbuf, sa, sb):
            for d in gdesc(b, rbuf, sa, sb):
                d.start()

        def gwait(b, rbuf, sa, sb):
            for d in gdesc(b, rbuf, sa, sb):
                d.wait()

        def scat64(b, rbuf):
            ca = pltpu.async_copy(
                rbuf.at[pl.ds(0, 32)], accum.at[ck.at[2 * b]], t0, add=True)
            cb = pltpu.async_copy(
                rbuf.at[pl.ds(32, 32)], accum.at[ck.at[2 * b + 1]], t1,
                add=True)
            ca.wait()
            cb.wait()

        def run_range(kb, ub, vkeys):
            def cbody(i, cnt):
                kv = keys[pl.ds(i * 16, 16)]
                gv = gvals[pl.ds(i * 16, 16)]
                m = (kv >= kb) & (kv < ub)
                mi = jnp.where(m, jnp.full((16,), 1, jnp.int32),
                               jnp.zeros((16,), jnp.int32))
                cs = plsc.cumsum(mi)
                pos = cs - mi + cnt
                plsc.store_scatter(ck, [pos // 32, pos % 32], kv - kb, mask=m)
                plsc.store_scatter(cg, [pos], gv, mask=m)
                return cnt + cs[15]

            cnt = lax.fori_loop(0, EPT // 16, cbody, 0)
            dump = jnp.full((16,), C, jnp.int32) + s
            zi = jnp.zeros((16,), jnp.int32)
            lanes = lax.iota(jnp.int32, 16)
            for j in range(4):
                pp = cnt + j * 16 + lanes
                plsc.store_scatter(ck, [pp // 32, pp % 32], dump)
                plsc.store_scatter(cg, [pp], zi)
            nb = (cnt + 63) // 64
            return nb

        def batch_loop(nb):
            pl.when(nb > 0)(lambda: gstart(0, rows0, g0a, g0b))

            def pbody(g, carry):
                b0 = 2 * g
                b1 = b0 + 1
                gwait(b0, rows0, g0a, g0b)
                pl.when(b1 < nb)(lambda: gstart(b1, rows1, g1a, g1b))
                scat64(b0, rows0)

                def second():
                    gwait(b1, rows1, g1a, g1b)
                    pl.when(b1 + 1 < nb)(
                        lambda: gstart(b1 + 1, rows0, g0a, g0b))
                    scat64(b1, rows1)

                pl.when(b1 < nb)(second)
                return carry

            lax.fori_loop(0, (nb + 1) // 2, pbody, 0)

        def compact_and_add(kb, ub, vkeys):
            batch_loop(run_range(kb, ub, vkeys))

        def copy_out(kb, vkeys):
            npt = vkeys // NS
            row0 = s * npt
            pltpu.sync_copy(
                accum.at[pl.ds(row0, npt)], out_hbm.at[pl.ds(kb + row0, npt)]
            )

        for p in range(npass):
            zcps = []
            off = 0
            while off < zrows:
                n = min(zrows - off, ZB)
                zcps.append(pltpu.async_copy(
                    zbuf.at[pl.ds(0, n)], accum.at[pl.ds(zlo + off, n)], zsem
                ))
                off += n
            for cv, (lo_k, hi_k) in enumerate(splits):
                kb = lo_k + p * C
                ub = min(kb + C, hi_k)
                if ub - kb <= 0:
                    continue

                def one_side(kb=kb, ub=ub):
                    nb = run_range(kb, ub, ub - kb)
                    for cp in zcps:
                        cp.wait()
                    plsc.subcore_barrier()
                    batch_loop(nb)

                pl.when(c == cv)(one_side)
            plsc.subcore_barrier()
            for cv, (lo_k, hi_k) in enumerate(splits):
                kb = lo_k + p * C
                ub = min(kb + C, hi_k)
                if ub - kb <= 0:
                    continue
                pl.when(c == cv)(functools.partial(copy_out, kb, ub - kb))
            plsc.subcore_barrier()

    return seg


_seg_pre = _make_segsum(
    W=D, table_rows=N,
    splits=((0, N * R // 2), (N * R // 2, N * R)), C=3040, npass=10,
)
_seg_att = _make_segsum(
    W=272, table_rows=E,
    splits=((0, 4992), (4992, N)), C=2512, npass=2,
)


def _make_edge_gather():
    EPW = E // (NC * NS)
    NB = EPW // 64
    TAIL = EPW - NB * 64

    @functools.partial(
        pl.kernel,
        mesh=_mesh(),
        compiler_params=pltpu.CompilerParams(use_tc_tiling_on_sc=False, needs_layout_passes=False),
        out_type=(
            jax.ShapeDtypeStruct((E, D), jnp.float32),
            jax.ShapeDtypeStruct((E, 2 * D), jnp.float32),
        ),
        scratch_types=[
            pltpu.VMEM((EPW,), jnp.int32),
            pltpu.VMEM((EPW,), jnp.int32),
            pltpu.VMEM((64, D), jnp.float32),
            pltpu.VMEM((64, D), jnp.float32),
            pltpu.VMEM((64, 2 * D), jnp.float32),
            pltpu.VMEM((64, 2 * D), jnp.float32),
            pltpu.SemaphoreType.DMA,
            pltpu.SemaphoreType.DMA,
            pltpu.SemaphoreType.DMA,
            pltpu.SemaphoreType.DMA,
        ],
    )
    def eg(q_hbm, kv_hbm, dst_hbm, src_hbm, qg_hbm, kvg_hbm,
           dstv, srcv, qr0, qr1, kvr0, kvr1, sq0, sq1, sk0, sk1):
        c = lax.axis_index("c")
        s = lax.axis_index("s")
        base = (s * NC + c) * EPW
        pltpu.sync_copy(dst_hbm.at[pl.ds(base, EPW)], dstv)
        pltpu.sync_copy(src_hbm.at[pl.ds(base, EPW)], srcv)

        qb = (qr0, qr1)
        kb = (kvr0, kvr1)
        qs = (sq0, sq1)
        ks = (sk0, sk1)
        sizes = [64] * NB + ([TAIL] if TAIL else [])

        def start(b):
            sz = sizes[b]
            pltpu.async_copy(
                q_hbm.at[dstv.at[pl.ds(b * 64, sz)]],
                qb[b % 2].at[pl.ds(0, sz)], qs[b % 2]
            )
            pltpu.async_copy(
                kv_hbm.at[srcv.at[pl.ds(b * 64, sz)]],
                kb[b % 2].at[pl.ds(0, sz)], ks[b % 2]
            )

        def drain(b):
            sz = sizes[b]
            pltpu.make_async_copy(
                q_hbm.at[dstv.at[pl.ds(b * 64, sz)]],
                qb[b % 2].at[pl.ds(0, sz)], qs[b % 2]
            ).wait()
            pltpu.make_async_copy(
                kv_hbm.at[srcv.at[pl.ds(b * 64, sz)]],
                kb[b % 2].at[pl.ds(0, sz)], ks[b % 2]
            ).wait()

        start(0)
        for b in range(len(sizes)):
            sz = sizes[b]
            if b + 1 < len(sizes):
                start(b + 1)
            drain(b)
            pltpu.sync_copy(
                qb[b % 2].at[pl.ds(0, sz)],
                qg_hbm.at[pl.ds(base + b * 64, sz)]
            )
            pltpu.sync_copy(
                kb[b % 2].at[pl.ds(0, sz)],
                kvg_hbm.at[pl.ds(base + b * 64, sz)]
            )

    return eg


_edge_gather = _make_edge_gather()

_BN2 = 400
_BE = 1000
_BN6 = 2000


def _qkv_body(x_ref, wbq, cq, wbk, ck_, wbv, cv_, bcat, q_ref, kv_ref):
    x = x_ref[...]
    acc = jnp.zeros((_BN2, 3 * D), jnp.float32)
    for r in range(R):
        mats = []
        for wb_ref, c_ref in ((wbq, cq), (wbk, ck_), (wbv, cv_)):
            comp = c_ref[...]
            wb = wb_ref[...]
            mats.append(jnp.sum(wb * comp[r, :, None, None], axis=0))
        wrc = jnp.concatenate(mats, axis=1)
        acc = acc + jnp.dot(
            x[:, r * D:(r + 1) * D], wrc, preferred_element_type=jnp.float32
        )
    y = jnp.maximum(acc + bcat[...], 0.0)
    q_ref[...] = y[:, :D]
    kv_ref[...] = y[:, D:]


def _qkv(pre2d, wbq, cq, wbk, ck_, wbv, cv_, bcat):
    full = lambda shape: pl.BlockSpec(shape, lambda i: (0,) * len(shape))
    return pl.pallas_call(
        _qkv_body,
        grid=(N // _BN2,),
        in_specs=[
            pl.BlockSpec((_BN2, R * D), lambda i: (i, 0)),
            full((R, D, D)), full((R, R)),
            full((R, D, D)), full((R, R)),
            full((R, D, D)), full((R, R)),
            full((1, 3 * D)),
        ],
        out_specs=(
            pl.BlockSpec((_BN2, D), lambda i: (i, 0)),
            pl.BlockSpec((_BN2, 2 * D), lambda i: (i, 0)),
        ),
        out_shape=(
            jax.ShapeDtypeStruct((N, D), jnp.float32),
            jax.ShapeDtypeStruct((N, 2 * D), jnp.float32),
        ),
    )(pre2d, wbq, cq, wbk, ck_, wbv, cv_, bcat)


def _score_body(qg_ref, kvg_ref, o_ref):
    q = qg_ref[...]
    kv = kvg_ref[...]
    k = kv[:, :D]
    v = kv[:, D:]
    ri = lax.broadcasted_iota(jnp.int32, (D, H), 0)
    ci = lax.broadcasted_iota(jnp.int32, (D, H), 1)
    bm = jnp.where(ri // (D // H) == ci, 1.0, 0.0)
    r2 = lax.broadcasted_iota(jnp.int32, (H, D), 0)
    c2 = lax.broadcasted_iota(jnp.int32, (H, D), 1)
    bm2 = jnp.where(c2 // (D // H) == r2, 1.0, 0.0)
    sc = jnp.dot(q * k, bm, preferred_element_type=jnp.float32)
    sc = jnp.exp(jnp.clip(sc * (1.0 / np.sqrt(D // H)), -10.0, 10.0))
    sw = jnp.dot(sc, bm2, preferred_element_type=jnp.float32)
    o_ref[...] = jnp.concatenate(
        [v * sw, sc, jnp.zeros((_BE, 8), jnp.float32)], axis=1
    )


def _scores(qg, kvg):
    return pl.pallas_call(
        _score_body,
        grid=(E // _BE,),
        in_specs=[
            pl.BlockSpec((_BE, D), lambda i: (i, 0)),
            pl.BlockSpec((_BE, 2 * D), lambda i: (i, 0)),
        ],
        out_specs=pl.BlockSpec((_BE, 272), lambda i: (i, 0)),
        out_shape=jax.ShapeDtypeStruct((E, 272), jnp.float32),
    )(qg, kvg)


def _divide_body(s_ref, o_ref):
    x = s_ref[...]
    wv = x[:, :D]
    z = x[:, D:D + H]
    r2 = lax.broadcasted_iota(jnp.int32, (H, D), 0)
    c2 = lax.broadcasted_iota(jnp.int32, (H, D), 1)
    bm2 = jnp.where(c2 // (D // H) == r2, 1.0, 0.0)
    zr = jnp.dot(z, bm2, preferred_element_type=jnp.float32)
    o_ref[...] = wv / (zr + 1e-6)


def _divide(sums):
    return pl.pallas_call(
        _divide_body,
        grid=(N // _BN6,),
        in_specs=[pl.BlockSpec((_BN6, 272), lambda i: (i, 0))],
        out_specs=pl.BlockSpec((_BN6, D), lambda i: (i, 0)),
        out_shape=jax.ShapeDtypeStruct((N, D), jnp.float32),
    )(sums)


def kernel(h, edge_index, e, Wb_Q, comp_Q, bias_Q,
           Wb_K, comp_K, bias_K, Wb_V, comp_V, bias_V):
    src = edge_index[0]
    dst = edge_index[1]
    keys1 = dst * R + e
    pre = _seg_pre(h, src, keys1)
    bcat = jnp.concatenate([bias_Q, bias_K, bias_V]).reshape(1, 3 * D)
    q, kv = _qkv(pre.reshape(N, R * D),
                 Wb_Q, comp_Q, Wb_K, comp_K, Wb_V, comp_V, bcat)
    qg, kvg = _edge_gather(q, kv, dst, src)
    contrib = _scores(qg, kvg)
    eidx = jnp.arange(E, dtype=jnp.int32)
    sums = _seg_att(contrib, eidx, dst)
    return _divide(sums).reshape(N, H, D // H)


# speedup vs baseline: 1.0347x; 1.0347x over previous
import functools

import jax
import jax.numpy as jnp
import numpy as np
from jax import lax
from jax.experimental import pallas as pl
from jax.experimental.pallas import tpu as pltpu
from jax.experimental.pallas import tpu_sc as plsc

N = 10000
E = 160000
D = 256
H = 8
R = 6
NS = 16
NC = 2

_mesh = functools.partial(
    plsc.VectorSubcoreMesh, core_axis_name="c", subcore_axis_name="s"
)


def _make_segsum(W, table_rows, splits, C, npass, packed=False):
    EPT = E // NS
    Cacc = C + NS
    CAP = EPT + 368
    ZB = 16
    KEYS = splits[-1][1]
    assert Cacc % NS == 0 and EPT % 16 == 0

    @functools.partial(
        pl.kernel,
        mesh=_mesh(),
        compiler_params=pltpu.CompilerParams(use_tc_tiling_on_sc=False, needs_layout_passes=False),
        out_type=jax.ShapeDtypeStruct((KEYS, W), jnp.float32),
        scratch_types=[
            pltpu.VMEM((EPT,), jnp.int32),
            pltpu.VMEM((16,) if packed else (EPT,), jnp.int32),
            pltpu.VMEM((CAP,), jnp.int32),
            pltpu.VMEM((CAP // 32, 32), jnp.int32),
            pltpu.VMEM((64, W), jnp.float32),
            pltpu.VMEM((64, W), jnp.float32),
            pltpu.VMEM((ZB, W), jnp.float32),
            pltpu.VMEM_SHARED((Cacc, W), jnp.float32),
            pltpu.SemaphoreType.DMA,
            pltpu.SemaphoreType.DMA,
            pltpu.SemaphoreType.DMA,
            pltpu.SemaphoreType.DMA,
            pltpu.SemaphoreType.DMA,
            pltpu.SemaphoreType.DMA,
            pltpu.SemaphoreType.DMA,
        ],
    )
    def seg(table_hbm, gv_hbm, key_hbm, out_hbm,
            gvals, keys, cg, ck, rows0, rows1, zbuf, accum,
            g0a, g0b, g1a, g1b, t0, t1, zsem):
        c = lax.axis_index("c")
        s = lax.axis_index("s")
        ebase = s * EPT
        pltpu.sync_copy(gv_hbm.at[pl.ds(ebase, EPT)], gvals)
        if not packed:
            pltpu.sync_copy(key_hbm.at[pl.ds(ebase, EPT)], keys)

        zv = jnp.zeros((16,), jnp.float32)
        wl = W // 16

        def zb_body(i, carry):
            zbuf[i // wl, pl.ds((i % wl) * 16, 16)] = zv
            return carry

        lax.fori_loop(0, ZB * wl, zb_body, 0)

        zrows = Cacc // NS
        zlo = s * zrows

        def gdesc(b, rbuf, sa, sb):
            return (
                pltpu.make_async_copy(
                    table_hbm.at[cg.at[pl.ds(b * 64, 32)]],
                    rbuf.at[pl.ds(0, 32)], sa),
                pltpu.make_async_copy(
                    table_hbm.at[cg.at[pl.ds(b * 64 + 32, 32)]],
                    rbuf.at[pl.ds(32, 32)], sb),
            )

        def gstart(b, rbuf, sa, sb):
            for d in gdesc(b, rbuf, sa, sb):
                d.start()

        def gwait(b, rbuf, sa, sb):
            for d in gdesc(b, rbuf, sa, sb):
                d.wait()

        def scat64(b, rbuf):
            ca = pltpu.async_copy(
                rbuf.at[pl.ds(0, 32)], accum.at[ck.at[2 * b]], t0, add=True)
            cb = pltpu.async_copy(
                rbuf.at[pl.ds(32, 32)], accum.at[ck.at[2 * b + 1]], t1,
                add=True)
            ca.wait()
            cb.wait()

        def run_range(kb, ub, vkeys):
            if packed:
                kbp = kb * 16384
                ubp = ub * 16384

                def cbody(i, cnt):
                    pk = gvals[pl.ds(i * 16, 16)]
                    m = (pk >= kbp) & (pk < ubp)
                    mi = jnp.where(m, jnp.full((16,), 1, jnp.int32),
                                   jnp.zeros((16,), jnp.int32))
                    cs = plsc.cumsum(mi)
                    pos = cs - mi + cnt
                    plsc.store_scatter(cg, [pos], pk, mask=m)
                    return cnt + cs[15]

                cnt = lax.fori_loop(0, EPT // 16, cbody, 0)

                def ubody(i, carry):
                    v = cg[pl.ds(i * 16, 16)]
                    cg[pl.ds(i * 16, 16)] = v & 16383
                    ck[i // 2, pl.ds((i % 2) * 16, 16)] = (v >> 14) - kb
                    return carry

                lax.fori_loop(0, (cnt + 15) // 16, ubody, 0)
            else:
                def cbody(i, cnt):
                    kv = keys[pl.ds(i * 16, 16)]
                    gv = gvals[pl.ds(i * 16, 16)]
                    m = (kv >= kb) & (kv < ub)
                    mi = jnp.where(m, jnp.full((16,), 1, jnp.int32),
                                   jnp.zeros((16,), jnp.int32))
                    cs = plsc.cumsum(mi)
                    pos = cs - mi + cnt
                    plsc.store_scatter(ck, [pos // 32, pos % 32], kv - kb,
                                       mask=m)
                    plsc.store_scatter(cg, [pos], gv, mask=m)
                    return cnt + cs[15]

                cnt = lax.fori_loop(0, EPT // 16, cbody, 0)
            dump = jnp.full((16,), C, jnp.int32) + s
            zi = jnp.zeros((16,), jnp.int32)
            lanes = lax.iota(jnp.int32, 16)
            for j in range(4):
                pp = cnt + j * 16 + lanes
                plsc.store_scatter(ck, [pp // 32, pp % 32], dump)
                plsc.store_scatter(cg, [pp], zi)
            nb = (cnt + 63) // 64
            return nb

        def batch_loop(nb):
            pl.when(nb > 0)(lambda: gstart(0, rows0, g0a, g0b))

            def pbody(g, carry):
                b0 = 2 * g
                b1 = b0 + 1
                gwait(b0, rows0, g0a, g0b)
                pl.when(b1 < nb)(lambda: gstart(b1, rows1, g1a, g1b))
                scat64(b0, rows0)

                def second():
                    gwait(b1, rows1, g1a, g1b)
                    pl.when(b1 + 1 < nb)(
                        lambda: gstart(b1 + 1, rows0, g0a, g0b))
                    scat64(b1, rows1)

                pl.when(b1 < nb)(second)
                return carry

            lax.fori_loop(0, (nb + 1) // 2, pbody, 0)

        def compact_and_add(kb, ub, vkeys):
            batch_loop(run_range(kb, ub, vkeys))

        def copy_out(kb, vkeys):
            npt = vkeys // NS
            row0 = s * npt
            pltpu.sync_copy(
                accum.at[pl.ds(row0, npt)], out_hbm.at[pl.ds(kb + row0, npt)]
            )

        for p in range(npass):
            zcps = []
            off = 0
            while off < zrows:
                n = min(zrows - off, ZB)
                zcps.append(pltpu.async_copy(
                    zbuf.at[pl.ds(0, n)], accum.at[pl.ds(zlo + off, n)], zsem
                ))
                off += n
            for cv, (lo_k, hi_k) in enumerate(splits):
                kb = lo_k + p * C
                ub = min(kb + C, hi_k)
                if ub - kb <= 0:
                    continue

                def one_side(kb=kb, ub=ub):
                    nb = run_range(kb, ub, ub - kb)
                    for cp in zcps:
                        cp.wait()
                    plsc.subcore_barrier()
                    batch_loop(nb)

                pl.when(c == cv)(one_side)
            plsc.subcore_barrier()
            for cv, (lo_k, hi_k) in enumerate(splits):
                kb = lo_k + p * C
                ub = min(kb + C, hi_k)
                if ub - kb <= 0:
                    continue
                pl.when(c == cv)(functools.partial(copy_out, kb, ub - kb))
            plsc.subcore_barrier()

    return seg


_seg_pre = _make_segsum(
    W=D, table_rows=N,
    splits=((0, N * R // 2), (N * R // 2, N * R)), C=3936, npass=8,
    packed=True,
)
_seg_att = _make_segsum(
    W=272, table_rows=E,
    splits=((0, 4992), (4992, N)), C=2512, npass=2,
)


def _make_edge_gather():
    EPW = E // (NC * NS)
    NB = EPW // 64
    TAIL = EPW - NB * 64

    @functools.partial(
        pl.kernel,
        mesh=_mesh(),
        compiler_params=pltpu.CompilerParams(use_tc_tiling_on_sc=False, needs_layout_passes=False),
        out_type=(
            jax.ShapeDtypeStruct((E, D), jnp.float32),
            jax.ShapeDtypeStruct((E, 2 * D), jnp.float32),
        ),
        scratch_types=[
            pltpu.VMEM((EPW,), jnp.int32),
            pltpu.VMEM((EPW,), jnp.int32),
            pltpu.VMEM((64, D), jnp.float32),
            pltpu.VMEM((64, D), jnp.float32),
            pltpu.VMEM((64, 2 * D), jnp.float32),
            pltpu.VMEM((64, 2 * D), jnp.float32),
            pltpu.SemaphoreType.DMA,
            pltpu.SemaphoreType.DMA,
            pltpu.SemaphoreType.DMA,
            pltpu.SemaphoreType.DMA,
        ],
    )
    def eg(q_hbm, kv_hbm, dst_hbm, src_hbm, qg_hbm, kvg_hbm,
           dstv, srcv, qr0, qr1, kvr0, kvr1, sq0, sq1, sk0, sk1):
        c = lax.axis_index("c")
        s = lax.axis_index("s")
        base = (s * NC + c) * EPW
        pltpu.sync_copy(dst_hbm.at[pl.ds(base, EPW)], dstv)
        pltpu.sync_copy(src_hbm.at[pl.ds(base, EPW)], srcv)

        qb = (qr0, qr1)
        kb = (kvr0, kvr1)
        qs = (sq0, sq1)
        ks = (sk0, sk1)
        sizes = [64] * NB + ([TAIL] if TAIL else [])

        def start(b):
            sz = sizes[b]
            pltpu.async_copy(
                q_hbm.at[dstv.at[pl.ds(b * 64, sz)]],
                qb[b % 2].at[pl.ds(0, sz)], qs[b % 2]
            )
            pltpu.async_copy(
                kv_hbm.at[srcv.at[pl.ds(b * 64, sz)]],
                kb[b % 2].at[pl.ds(0, sz)], ks[b % 2]
            )

        def drain(b):
            sz = sizes[b]
            pltpu.make_async_copy(
                q_hbm.at[dstv.at[pl.ds(b * 64, sz)]],
                qb[b % 2].at[pl.ds(0, sz)], qs[b % 2]
            ).wait()
            pltpu.make_async_copy(
                kv_hbm.at[srcv.at[pl.ds(b * 64, sz)]],
                kb[b % 2].at[pl.ds(0, sz)], ks[b % 2]
            ).wait()

        start(0)
        for b in range(len(sizes)):
            sz = sizes[b]
            if b + 1 < len(sizes):
                start(b + 1)
            drain(b)
            pltpu.sync_copy(
                qb[b % 2].at[pl.ds(0, sz)],
                qg_hbm.at[pl.ds(base + b * 64, sz)]
            )
            pltpu.sync_copy(
                kb[b % 2].at[pl.ds(0, sz)],
                kvg_hbm.at[pl.ds(base + b * 64, sz)]
            )

    return eg


_edge_gather = _make_edge_gather()

_BN2 = 400
_BE = 1000
_BN6 = 2000


def _qkv_body(x_ref, wbq, cq, wbk, ck_, wbv, cv_, bcat, q_ref, kv_ref):
    x = x_ref[...]
    acc = jnp.zeros((_BN2, 3 * D), jnp.float32)
    for r in range(R):
        mats = []
        for wb_ref, c_ref in ((wbq, cq), (wbk, ck_), (wbv, cv_)):
            comp = c_ref[...]
            wb = wb_ref[...]
            mats.append(jnp.sum(wb * comp[r, :, None, None], axis=0))
        wrc = jnp.concatenate(mats, axis=1)
        acc = acc + jnp.dot(
            x[:, r * D:(r + 1) * D], wrc, preferred_element_type=jnp.float32
        )
    y = jnp.maximum(acc + bcat[...], 0.0)
    q_ref[...] = y[:, :D]
    kv_ref[...] = y[:, D:]


def _qkv(pre2d, wbq, cq, wbk, ck_, wbv, cv_, bcat):
    full = lambda shape: pl.BlockSpec(shape, lambda i: (0,) * len(shape))
    return pl.pallas_call(
        _qkv_body,
        grid=(N // _BN2,),
        in_specs=[
            pl.BlockSpec((_BN2, R * D), lambda i: (i, 0)),
            full((R, D, D)), full((R, R)),
            full((R, D, D)), full((R, R)),
            full((R, D, D)), full((R, R)),
            full((1, 3 * D)),
        ],
        out_specs=(
            pl.BlockSpec((_BN2, D), lambda i: (i, 0)),
            pl.BlockSpec((_BN2, 2 * D), lambda i: (i, 0)),
        ),
        out_shape=(
            jax.ShapeDtypeStruct((N, D), jnp.float32),
            jax.ShapeDtypeStruct((N, 2 * D), jnp.float32),
        ),
    )(pre2d, wbq, cq, wbk, ck_, wbv, cv_, bcat)


def _score_body(qg_ref, kvg_ref, o_ref):
    q = qg_ref[...]
    kv = kvg_ref[...]
    k = kv[:, :D]
    v = kv[:, D:]
    ri = lax.broadcasted_iota(jnp.int32, (D, H), 0)
    ci = lax.broadcasted_iota(jnp.int32, (D, H), 1)
    bm = jnp.where(ri // (D // H) == ci, 1.0, 0.0)
    r2 = lax.broadcasted_iota(jnp.int32, (H, D), 0)
    c2 = lax.broadcasted_iota(jnp.int32, (H, D), 1)
    bm2 = jnp.where(c2 // (D // H) == r2, 1.0, 0.0)
    sc = jnp.dot(q * k, bm, preferred_element_type=jnp.float32)
    sc = jnp.exp(jnp.clip(sc * (1.0 / np.sqrt(D // H)), -10.0, 10.0))
    sw = jnp.dot(sc, bm2, preferred_element_type=jnp.float32)
    o_ref[...] = jnp.concatenate(
        [v * sw, sc, jnp.zeros((_BE, 8), jnp.float32)], axis=1
    )


def _scores(qg, kvg):
    return pl.pallas_call(
        _score_body,
        grid=(E // _BE,),
        in_specs=[
            pl.BlockSpec((_BE, D), lambda i: (i, 0)),
            pl.BlockSpec((_BE, 2 * D), lambda i: (i, 0)),
        ],
        out_specs=pl.BlockSpec((_BE, 272), lambda i: (i, 0)),
        out_shape=jax.ShapeDtypeStruct((E, 272), jnp.float32),
    )(qg, kvg)


def _divide_body(s_ref, o_ref):
    x = s_ref[...]
    wv = x[:, :D]
    z = x[:, D:D + H]
    r2 = lax.broadcasted_iota(jnp.int32, (H, D), 0)
    c2 = lax.broadcasted_iota(jnp.int32, (H, D), 1)
    bm2 = jnp.where(c2 // (D // H) == r2, 1.0, 0.0)
    zr = jnp.dot(z, bm2, preferred_element_type=jnp.float32)
    o_ref[...] = wv / (zr + 1e-6)


def _divide(sums):
    return pl.pallas_call(
        _divide_body,
        grid=(N // _BN6,),
        in_specs=[pl.BlockSpec((_BN6, 272), lambda i: (i, 0))],
        out_specs=pl.BlockSpec((_BN6, D), lambda i: (i, 0)),
        out_shape=jax.ShapeDtypeStruct((N, D), jnp.float32),
    )(sums)


def kernel(h, edge_index, e, Wb_Q, comp_Q, bias_Q,
           Wb_K, comp_K, bias_K, Wb_V, comp_V, bias_V):
    src = edge_index[0]
    dst = edge_index[1]
    keys1 = dst * R + e
    packed1 = keys1 * 16384 + src
    pre = _seg_pre(h, packed1, packed1)
    bcat = jnp.concatenate([bias_Q, bias_K, bias_V]).reshape(1, 3 * D)
    q, kv = _qkv(pre.reshape(N, R * D),
                 Wb_Q, comp_Q, Wb_K, comp_K, Wb_V, comp_V, bcat)
    qg, kvg = _edge_gather(q, kv, dst, src)
    contrib = _scores(qg, kvg)
    eidx = jnp.arange(E, dtype=jnp.int32)
    sums = _seg_att(contrib, eidx, dst)
    return _divide(sums).reshape(N, H, D // H)
